# Initial kernel scaffold; baseline (speedup 1.0000x reference)
import jax
import jax.numpy as jnp
from jax.experimental import pallas as pl


def kernel(x_list, edge_index, edge_weight, W1, b1, W2, b2):
    raise NotImplementedError("write your pallas kernel here")


# speedup vs baseline: 3.3550x; 3.3550x over previous
import functools

import jax
import jax.numpy as jnp
from jax import lax
from jax.experimental import pallas as pl
from jax.experimental.pallas import tpu as pltpu
from jax.experimental.pallas import tpu_sc as plsc

N = 10000
NP = 10240
E = 320000
C = 4
D = 128
HID = 128
K = 40

NC = 2
NS = 16
LANES = 16

EDGES_PER_TILE = E // NS
GROUP = 80
GROUPS = EDGES_PER_TILE // GROUP
ROWS_PER_TILE = NP // NS


def _sc_smooth(h, src, dst, w, width, chunks_per_core):
  n_chunks = NC * chunks_per_core
  subvecs = width // LANES

  mesh = plsc.VectorSubcoreMesh(core_axis_name="c", subcore_axis_name="s")

  @functools.partial(
      pl.kernel,
      mesh=mesh,
      compiler_params=pltpu.CompilerParams(
          use_tc_tiling_on_sc=(width % 128 == 0)),
      out_type=jax.ShapeDtypeStruct((n_chunks, NP, width), jnp.float32),
      scratch_types=[
          pltpu.VMEM((GROUP,), jnp.int32),
          pltpu.VMEM((GROUP,), jnp.int32),
          pltpu.VMEM((GROUP,), jnp.float32),
          pltpu.VMEM((GROUP, width), jnp.float32),
          pltpu.VMEM_SHARED((NP, width), jnp.float32),
          pltpu.SemaphoreType.DMA,
      ],
  )
  def smooth(h_hbm, src_hbm, dst_hbm, w_hbm, zeros_hbm, out_hbm,
             src_v, dst_v, w_v, rows_v, acc, sem):
    ci = lax.axis_index("c")
    si = lax.axis_index("s")
    row0 = si * ROWS_PER_TILE

    for cc in range(chunks_per_core):
      chunk = ci * chunks_per_core + cc
      pltpu.sync_copy(zeros_hbm.at[pl.ds(row0, ROWS_PER_TILE)],
                      acc.at[pl.ds(row0, ROWS_PER_TILE)])
      plsc.subcore_barrier()

      def group_body(g, carry):
        base = si * EDGES_PER_TILE + g * GROUP
        pltpu.sync_copy(src_hbm.at[pl.ds(base, GROUP)], src_v)
        pltpu.sync_copy(dst_hbm.at[pl.ds(base, GROUP)], dst_v)
        pltpu.sync_copy(w_hbm.at[pl.ds(base, GROUP)], w_v)
        pltpu.async_copy(h_hbm.at[chunk].at[src_v], rows_v, sem).wait()

        def scale_body(g16, carry2):
          e0 = g16 * LANES
          wv16 = w_v[pl.ds(e0, LANES)]
          for j in range(LANES):
            wj = wv16[j]
            for sv in range(subvecs):
              sl = pl.ds(sv * LANES, LANES)
              rows_v[e0 + j, sl] = rows_v[e0 + j, sl] * wj
          return carry2

        lax.fori_loop(0, GROUP // LANES, scale_body, 0)
        pltpu.sync_copy(rows_v, acc.at[dst_v], add=True)
        return carry

      lax.fori_loop(0, GROUPS, group_body, 0)
      plsc.subcore_barrier()
      pltpu.sync_copy(acc.at[pl.ds(row0, ROWS_PER_TILE)],
                      out_hbm.at[chunk].at[pl.ds(row0, ROWS_PER_TILE)])
      plsc.subcore_barrier()

  zeros = jnp.zeros((NP, width), jnp.float32)
  return smooth(h, src, dst, w, zeros)


def _tc_layer1(x_list, w1, b1):
  blk = 640

  def body(x_ref, w_ref, b_ref, o_ref):
    o_ref[0] = (jnp.dot(x_ref[0], w_ref[...],
                        preferred_element_type=jnp.float32) + b_ref[...])

  return pl.pallas_call(
      body,
      grid=(C, NP // blk),
      in_specs=[
          pl.BlockSpec((1, blk, D), lambda c, i: (c, i, 0)),
          pl.BlockSpec((D, HID), lambda c, i: (0, 0)),
          pl.BlockSpec((1, HID), lambda c, i: (0, 0)),
      ],
      out_specs=pl.BlockSpec((1, blk, HID), lambda c, i: (c, i, 0)),
      out_shape=jax.ShapeDtypeStruct((C, NP, HID), jnp.float32),
  )(x_list, w1, b1.reshape(1, HID))


def _tc_layer2(s1, w2p, b2p):
  blk = 640

  def body(s_ref, w_ref, b_ref, o_ref):
    acc = jnp.broadcast_to(b_ref[...], (blk, 64))
    for c in range(C):
      acc = acc + jnp.dot(jnp.maximum(s_ref[c], 0.0), w_ref[c],
                          preferred_element_type=jnp.float32)
    o_ref[0] = acc[:, :32]
    o_ref[1] = acc[:, 32:]

  return pl.pallas_call(
      body,
      grid=(NP // blk,),
      in_specs=[
          pl.BlockSpec((C, blk, HID), lambda i: (0, i, 0)),
          pl.BlockSpec((C, HID, 64), lambda i: (0, 0, 0)),
          pl.BlockSpec((1, 64), lambda i: (0, 0)),
      ],
      out_specs=pl.BlockSpec((2, blk, 32), lambda i: (0, i, 0)),
      out_shape=jax.ShapeDtypeStruct((2, NP, 32), jnp.float32),
  )(s1, w2p, b2p)


def kernel(x_list, edge_index, edge_weight, W1, b1, W2, b2):
  src = edge_index[0]
  dst = edge_index[1]

  xp = jnp.pad(x_list, ((0, 0), (0, NP - N), (0, 0)))
  hlin = _tc_layer1(xp, W1, b1)
  s1 = _sc_smooth(hlin, src, dst, edge_weight, HID, chunks_per_core=2)

  w2p = jnp.pad(W2, ((0, 0), (0, 64 - K))).reshape(C, HID, 64)
  b2p = jnp.pad(b2, (0, 64 - K)).reshape(1, 64)
  zp = _tc_layer2(s1, w2p, b2p)

  s2 = _sc_smooth(zp, src, dst, edge_weight, 32, chunks_per_core=1)
  return jnp.concatenate([s2[0, :N], s2[1, :N, : K - 32]], axis=1)


# speedup vs baseline: 9.1087x; 2.7150x over previous
import functools

import jax
import jax.numpy as jnp
from jax import lax
from jax.experimental import pallas as pl
from jax.experimental.pallas import tpu as pltpu
from jax.experimental.pallas import tpu_sc as plsc

N = 10000
NP = 10240
E = 320000
C = 4
D = 128
HID = 128
K = 40

NC = 2
NS = 16
LANES = 16

EDGES_PER_TILE = E // NS
GROUP = 80
GROUPS = EDGES_PER_TILE // GROUP
NBLK = 5
BG = GROUPS // NBLK
ROWS_PER_TILE = NP // NS


def _sc_smooth(h, src, dst, w, width, chunks_per_core):
  n_chunks = NC * chunks_per_core
  subvecs = width // LANES

  mesh = plsc.VectorSubcoreMesh(core_axis_name="c", subcore_axis_name="s")

  @functools.partial(
      pl.kernel,
      mesh=mesh,
      compiler_params=pltpu.CompilerParams(
          use_tc_tiling_on_sc=(width % 128 == 0)),
      out_type=jax.ShapeDtypeStruct((n_chunks, NP, width), jnp.float32),
      scratch_types=[
          pltpu.VMEM((BG, GROUP), jnp.int32),
          pltpu.VMEM((BG, GROUP), jnp.int32),
          pltpu.VMEM((BG, GROUP), jnp.float32),
          pltpu.VMEM((2, GROUP, width), jnp.float32),
          pltpu.VMEM_SHARED((NP, width), jnp.float32),
          [pltpu.SemaphoreType.DMA] * 2,
          [pltpu.SemaphoreType.DMA] * 2,
      ],
  )
  def smooth(h_hbm, src_hbm, dst_hbm, w_hbm, zeros_hbm, out_hbm,
             src_v, dst_v, w_v, rows_v, acc, sg, ss):
    ci = lax.axis_index("c")
    si = lax.axis_index("s")
    row0 = si * ROWS_PER_TILE

    def scale(b, g):
      def scale_body(g16, carry2):
        e0 = g16 * LANES
        wv16 = w_v[g, pl.ds(e0, LANES)]
        for j in range(LANES):
          wj = wv16[j]
          for sv in range(subvecs):
            sl = pl.ds(sv * LANES, LANES)
            rows_v[b, e0 + j, sl] = rows_v[b, e0 + j, sl] * wj
        return carry2

      lax.fori_loop(0, GROUP // LANES, scale_body, 0)

    for cc in range(chunks_per_core):
      chunk = ci * chunks_per_core + cc
      hc = h_hbm.at[chunk]
      pltpu.sync_copy(zeros_hbm.at[pl.ds(row0, ROWS_PER_TILE)],
                      acc.at[pl.ds(row0, ROWS_PER_TILE)])
      plsc.subcore_barrier()

      def gather(g, b):
        return pltpu.async_copy(hc.at[src_v.at[g]], rows_v.at[b], sg[b])

      def scatter(g, b):
        return pltpu.async_copy(rows_v.at[b], acc.at[dst_v.at[g]], ss[b],
                                add=True)

      def block_body(blk, carry):
        pltpu.sync_copy(src_hbm.at[si].at[blk], src_v)
        pltpu.sync_copy(dst_hbm.at[si].at[blk], dst_v)
        pltpu.sync_copy(w_hbm.at[si].at[blk], w_v)

        g0 = gather(0, 0)
        gather(1, 1)
        g0.wait()
        scale(0, 0)
        scatter(0, 0)

        def steady(g2, carry2):
          g = 2 * g2 + 1
          for b, dg in ((1, 1), (0, 2)):
            pltpu.make_async_copy(rows_v.at[1 - b], acc.at[dst_v.at[g]],
                                  ss[1 - b]).wait()
            gather(g + dg, 1 - b)
            pltpu.make_async_copy(hc.at[src_v.at[g]], rows_v.at[b],
                                  sg[b]).wait()
            scale(b, g + dg - 1)
            scatter(g + dg - 1, b)
          return carry2

        lax.fori_loop(0, (BG - 2) // 2, steady, 0)

        g_last = BG - 1
        pltpu.make_async_copy(hc.at[src_v.at[g_last]], rows_v.at[1],
                              sg[1]).wait()
        scale(1, g_last)
        scatter(g_last, 1)
        pltpu.make_async_copy(rows_v.at[0], acc.at[dst_v.at[g_last]],
                              ss[0]).wait()
        pltpu.make_async_copy(rows_v.at[1], acc.at[dst_v.at[g_last]],
                              ss[1]).wait()
        return carry

      lax.fori_loop(0, NBLK, block_body, 0)

      plsc.subcore_barrier()
      pltpu.sync_copy(acc.at[pl.ds(row0, ROWS_PER_TILE)],
                      out_hbm.at[chunk].at[pl.ds(row0, ROWS_PER_TILE)])
      plsc.subcore_barrier()

  zeros = jnp.zeros((NP, width), jnp.float32)
  src4 = src.reshape(NS, NBLK, BG, GROUP)
  dst4 = dst.reshape(NS, NBLK, BG, GROUP)
  w4 = w.reshape(NS, NBLK, BG, GROUP)
  return smooth(h, src4, dst4, w4, zeros)


def _tc_layer1(x_list, w1, b1):
  blk = 640

  def body(x_ref, w_ref, b_ref, o_ref):
    o_ref[0] = (jnp.dot(x_ref[0], w_ref[...],
                        preferred_element_type=jnp.float32) + b_ref[...])

  return pl.pallas_call(
      body,
      grid=(C, NP // blk),
      in_specs=[
          pl.BlockSpec((1, blk, D), lambda c, i: (c, i, 0)),
          pl.BlockSpec((D, HID), lambda c, i: (0, 0)),
          pl.BlockSpec((1, HID), lambda c, i: (0, 0)),
      ],
      out_specs=pl.BlockSpec((1, blk, HID), lambda c, i: (c, i, 0)),
      out_shape=jax.ShapeDtypeStruct((C, NP, HID), jnp.float32),
  )(x_list, w1, b1.reshape(1, HID))


def _tc_layer2(s1, w2p, b2p):
  blk = 640

  def body(s_ref, w_ref, b_ref, o_ref):
    acc = jnp.broadcast_to(b_ref[...], (blk, 64))
    for c in range(C):
      acc = acc + jnp.dot(jnp.maximum(s_ref[c], 0.0), w_ref[c],
                          preferred_element_type=jnp.float32)
    o_ref[0] = acc[:, :32]
    o_ref[1] = acc[:, 32:]

  return pl.pallas_call(
      body,
      grid=(NP // blk,),
      in_specs=[
          pl.BlockSpec((C, blk, HID), lambda i: (0, i, 0)),
          pl.BlockSpec((C, HID, 64), lambda i: (0, 0, 0)),
          pl.BlockSpec((1, 64), lambda i: (0, 0)),
      ],
      out_specs=pl.BlockSpec((2, blk, 32), lambda i: (0, i, 0)),
      out_shape=jax.ShapeDtypeStruct((2, NP, 32), jnp.float32),
  )(s1, w2p, b2p)


def kernel(x_list, edge_index, edge_weight, W1, b1, W2, b2):
  src = edge_index[0]
  dst = edge_index[1]

  xp = jnp.pad(x_list, ((0, 0), (0, NP - N), (0, 0)))
  hlin = _tc_layer1(xp, W1, b1)
  s1 = _sc_smooth(hlin, src, dst, edge_weight, HID, chunks_per_core=2)

  w2p = jnp.pad(W2, ((0, 0), (0, 64 - K))).reshape(C, HID, 64)
  b2p = jnp.pad(b2, (0, 64 - K)).reshape(1, 64)
  zp = _tc_layer2(s1, w2p, b2p)

  s2 = _sc_smooth(zp, src, dst, edge_weight, 32, chunks_per_core=1)
  return jnp.concatenate([s2[0, :N], s2[1, :N, : K - 32]], axis=1)


# speedup vs baseline: 9.1568x; 1.0053x over previous
import functools

import jax
import jax.numpy as jnp
from jax import lax
from jax.experimental import pallas as pl
from jax.experimental.pallas import tpu as pltpu
from jax.experimental.pallas import tpu_sc as plsc

N = 10000
NP = 10240
E = 320000
C = 4
D = 128
HID = 128
K = 40

NC = 2
NS = 16
LANES = 16

EDGES_PER_TILE = E // NS
GROUP = 80
GROUPS = EDGES_PER_TILE // GROUP
NBLK = 5
BG = GROUPS // NBLK
ROWS_PER_TILE = NP // NS


def _sc_smooth(h, src, dst, w, width, chunks_per_core):
  n_chunks = NC * chunks_per_core
  subvecs = width // LANES

  mesh = plsc.VectorSubcoreMesh(core_axis_name="c", subcore_axis_name="s")

  @functools.partial(
      pl.kernel,
      mesh=mesh,
      compiler_params=pltpu.CompilerParams(
          use_tc_tiling_on_sc=(width % 128 == 0)),
      out_type=jax.ShapeDtypeStruct((n_chunks, NP, width), jnp.float32),
      scratch_types=[
          pltpu.VMEM((BG, GROUP), jnp.int32),
          pltpu.VMEM((BG, GROUP), jnp.int32),
          pltpu.VMEM((BG, GROUP), jnp.float32),
          pltpu.VMEM((2, GROUP, width), jnp.float32),
          pltpu.VMEM_SHARED((NP, width), jnp.float32),
          [pltpu.SemaphoreType.DMA] * 2,
          [pltpu.SemaphoreType.DMA] * 2,
      ],
  )
  def smooth(h_hbm, src_hbm, dst_hbm, w_hbm, zeros_hbm, out_hbm,
             src_v, dst_v, w_v, rows_v, acc, sg, ss):
    ci = lax.axis_index("c")
    si = lax.axis_index("s")
    row0 = si * ROWS_PER_TILE

    def scale(b, g):
      @plsc.parallel_loop(0, GROUP // LANES)
      def scale_body(g16):
        e0 = g16 * LANES
        wv16 = w_v[g, pl.ds(e0, LANES)]
        for j in range(LANES):
          wj = wv16[j]
          for sv in range(subvecs):
            sl = pl.ds(sv * LANES, LANES)
            rows_v[b, e0 + j, sl] = rows_v[b, e0 + j, sl] * wj

    for cc in range(chunks_per_core):
      chunk = ci * chunks_per_core + cc
      hc = h_hbm.at[chunk]
      pltpu.sync_copy(zeros_hbm.at[pl.ds(row0, ROWS_PER_TILE)],
                      acc.at[pl.ds(row0, ROWS_PER_TILE)])
      plsc.subcore_barrier()

      def gather(g, b):
        return pltpu.async_copy(hc.at[src_v.at[g]], rows_v.at[b], sg[b])

      def scatter(g, b):
        return pltpu.async_copy(rows_v.at[b], acc.at[dst_v.at[g]], ss[b],
                                add=True)

      def block_body(blk, carry):
        pltpu.sync_copy(src_hbm.at[si].at[blk], src_v)
        pltpu.sync_copy(dst_hbm.at[si].at[blk], dst_v)
        pltpu.sync_copy(w_hbm.at[si].at[blk], w_v)

        g0 = gather(0, 0)
        gather(1, 1)
        g0.wait()
        scale(0, 0)
        scatter(0, 0)

        def steady(g2, carry2):
          g = 2 * g2 + 1
          for b, dg in ((1, 1), (0, 2)):
            pltpu.make_async_copy(rows_v.at[1 - b], acc.at[dst_v.at[g]],
                                  ss[1 - b]).wait()
            gather(g + dg, 1 - b)
            pltpu.make_async_copy(hc.at[src_v.at[g]], rows_v.at[b],
                                  sg[b]).wait()
            scale(b, g + dg - 1)
            scatter(g + dg - 1, b)
          return carry2

        lax.fori_loop(0, (BG - 2) // 2, steady, 0)

        g_last = BG - 1
        pltpu.make_async_copy(hc.at[src_v.at[g_last]], rows_v.at[1],
                              sg[1]).wait()
        scale(1, g_last)
        scatter(g_last, 1)
        pltpu.make_async_copy(rows_v.at[0], acc.at[dst_v.at[g_last]],
                              ss[0]).wait()
        pltpu.make_async_copy(rows_v.at[1], acc.at[dst_v.at[g_last]],
                              ss[1]).wait()
        return carry

      lax.fori_loop(0, NBLK, block_body, 0)

      plsc.subcore_barrier()
      pltpu.sync_copy(acc.at[pl.ds(row0, ROWS_PER_TILE)],
                      out_hbm.at[chunk].at[pl.ds(row0, ROWS_PER_TILE)])
      plsc.subcore_barrier()

  zeros = jnp.zeros((NP, width), jnp.float32)
  src4 = src.reshape(NS, NBLK, BG, GROUP)
  dst4 = dst.reshape(NS, NBLK, BG, GROUP)
  w4 = w.reshape(NS, NBLK, BG, GROUP)
  return smooth(h, src4, dst4, w4, zeros)


def _tc_layer1(x_list, w1, b1):
  blk = 640

  def body(x_ref, w_ref, b_ref, o_ref):
    o_ref[0] = (jnp.dot(x_ref[0], w_ref[...],
                        preferred_element_type=jnp.float32) + b_ref[...])

  return pl.pallas_call(
      body,
      grid=(C, NP // blk),
      in_specs=[
          pl.BlockSpec((1, blk, D), lambda c, i: (c, i, 0)),
          pl.BlockSpec((D, HID), lambda c, i: (0, 0)),
          pl.BlockSpec((1, HID), lambda c, i: (0, 0)),
      ],
      out_specs=pl.BlockSpec((1, blk, HID), lambda c, i: (c, i, 0)),
      out_shape=jax.ShapeDtypeStruct((C, NP, HID), jnp.float32),
  )(x_list, w1, b1.reshape(1, HID))


def _tc_layer2(s1, w2p, b2p):
  blk = 640

  def body(s_ref, w_ref, b_ref, o_ref):
    acc = jnp.broadcast_to(b_ref[...], (blk, 64))
    for c in range(C):
      acc = acc + jnp.dot(jnp.maximum(s_ref[c], 0.0), w_ref[c],
                          preferred_element_type=jnp.float32)
    o_ref[0] = acc[:, :32]
    o_ref[1] = acc[:, 32:]

  return pl.pallas_call(
      body,
      grid=(NP // blk,),
      in_specs=[
          pl.BlockSpec((C, blk, HID), lambda i: (0, i, 0)),
          pl.BlockSpec((C, HID, 64), lambda i: (0, 0, 0)),
          pl.BlockSpec((1, 64), lambda i: (0, 0)),
      ],
      out_specs=pl.BlockSpec((2, blk, 32), lambda i: (0, i, 0)),
      out_shape=jax.ShapeDtypeStruct((2, NP, 32), jnp.float32),
  )(s1, w2p, b2p)


def kernel(x_list, edge_index, edge_weight, W1, b1, W2, b2):
  src = edge_index[0]
  dst = edge_index[1]

  xp = jnp.pad(x_list, ((0, 0), (0, NP - N), (0, 0)))
  hlin = _tc_layer1(xp, W1, b1)
  s1 = _sc_smooth(hlin, src, dst, edge_weight, HID, chunks_per_core=2)

  w2p = jnp.pad(W2, ((0, 0), (0, 64 - K))).reshape(C, HID, 64)
  b2p = jnp.pad(b2, (0, 64 - K)).reshape(1, 64)
  zp = _tc_layer2(s1, w2p, b2p)

  s2 = _sc_smooth(zp, src, dst, edge_weight, 32, chunks_per_core=1)
  return jnp.concatenate([s2[0, :N], s2[1, :N, : K - 32]], axis=1)
